# baseline (speedup 1.0000x reference)
import functools

import jax
import jax.numpy as jnp
from jax import lax
from jax.experimental import pallas as pl
from jax.experimental.pallas import tpu as pltpu
from jax.experimental.pallas import tpu_sc as plsc

B = 16384
D = 16
F = 2 * D
NC = 2
NS = 16
NW = NC * NS
BPW = B // NW
CHUNK = 128
NCHUNK = BPW // CHUNK
NCOL = 7813
CPW = 245

LANES = 128
EPR = LANES // F
RB = B * F // LANES
GRID = 8
BLK = RB // GRID
RING = 64


def _sc_gather(users, movies, tab_ut, tab_mt, bu_t, bm_t):
  mesh = plsc.VectorSubcoreMesh(core_axis_name="c", subcore_axis_name="s")
  f32 = jnp.float32
  i32 = jnp.int32

  @functools.partial(
      pl.kernel,
      mesh=mesh,
      compiler_params=pltpu.CompilerParams(needs_layout_passes=False),
      out_type=[
          jax.ShapeDtypeStruct((B * F,), f32),
          jax.ShapeDtypeStruct((B * F,), f32),
          jax.ShapeDtypeStruct((B,), f32),
          jax.ShapeDtypeStruct((B,), f32),
      ],
      scratch_types=[
          pltpu.VMEM((B,), i32),
          pltpu.VMEM((B,), i32),
          pltpu.VMEM((2, F, LANES), f32),
          pltpu.VMEM((RING * F,), f32),
          pltpu.VMEM((BPW,), f32),
          pltpu.SemaphoreType.DMA,
          pltpu.SemaphoreType.DMA,
          pltpu.SemaphoreType.DMA,
          pltpu.SemaphoreType.DMA,
      ],
  )
  def k(us_h, mv_h, tu_h, tm_h, bu_h, bm_h,
        ur_o, mr_o, bug_o, bmg_o,
        idx_v, wl_v, tiles_v, rows_v, bias_v, sem, bsem, fsem0, fsem1):
    wid = lax.axis_index("s") * NC + lax.axis_index("c")
    cb_lo = wid * CPW
    cb_hi = jnp.minimum(cb_lo + CPW, NCOL)
    elo = cb_lo * LANES
    ehi = cb_hi * LANES
    lane16 = lax.iota(i32, 16)

    def do_table(idx_h, tab_h, out_o, bias_h, bias_o):
      pltpu.async_copy(idx_h, idx_v, bsem).wait()

      base = wid * BPW
      bias_hs = []
      for c in range(NCHUNK):
        src = bias_h.at[idx_v.at[pl.ds(pl.multiple_of(base + c * CHUNK, CHUNK), CHUNK)]]
        bias_hs.append(pltpu.async_copy(src, bias_v.at[pl.ds(c * CHUNK, CHUNK)], bsem))

      def compact(c, cnt):
        bvec = c * 16 + lane16
        uvec = idx_v[pl.ds(pl.multiple_of(c * 16, 16), 16)]
        mask = (uvec >= elo) & (uvec < ehi)
        mi = jnp.where(mask, 1, 0)
        pos = cnt + plsc.cumsum(mi) - 1
        plsc.store_scatter(wl_v, [pos], bvec, mask=mask)
        return cnt + jnp.sum(mi)
      cnt = lax.fori_loop(0, B // 16, compact, jnp.int32(0))
      nchunks = (cnt + 15) // 16

      pltpu.async_copy(
          tab_h.at[:, pl.ds(pl.multiple_of(cb_lo * LANES, LANES), LANES)],
          tiles_v.at[0], fsem0)

      def process_column(cb, slot, fsem, total):
        lo = cb * LANES
        hi = lo + LANES

        def scan_members(t, tot):
          valid = (t * 16 + lane16) < cnt
          bvec = wl_v[pl.ds(pl.multiple_of(t * 16, 16), 16)]
          uvec = plsc.load_gather(idx_v, [bvec], mask=valid)
          incol = valid & (uvec >= lo) & (uvec < hi)
          mi = jnp.where(incol, 1, 0)
          nm = jnp.sum(mi)

          @pl.when(nm > 0)
          def _():
            slots = tot + plsc.cumsum(mi) - 1
            for j in range(16):
              @pl.when(mi[j] == 1)
              def _():
                u = uvec[j]
                b = bvec[j]
                s = slots[j]
                ring = lax.rem(s, RING)
                ro = pl.multiple_of(ring * F, F)
                @pl.when(s >= RING)
                def _():
                  pltpu.make_async_copy(
                      out_o.at[pl.ds(0, F)],
                      rows_v.at[pl.ds(ro, F)], sem).wait()
                l = u - lo
                lv = jnp.full((16,), l, i32)
                va = plsc.load_gather(tiles_v.at[slot], [lane16, lv])
                vb = plsc.load_gather(tiles_v.at[slot], [lane16 + D, lv])
                rows_v[pl.ds(ro, D)] = va
                rows_v[pl.ds(pl.multiple_of(ring * F + D, D), D)] = vb
                pltpu.async_copy(
                    rows_v.at[pl.ds(ro, F)],
                    out_o.at[pl.ds(pl.multiple_of(b * F, F), F)], sem)
          return tot + nm

        return lax.fori_loop(0, nchunks, scan_members, total)

      def per_pair(p, total):
        cb0 = cb_lo + 2 * p
        cb1 = cb0 + 1
        @pl.when(cb1 < cb_hi)
        def _():
          nxt = pl.multiple_of(cb1 * LANES, LANES)
          pltpu.async_copy(tab_h.at[:, pl.ds(nxt, LANES)], tiles_v.at[1], fsem1)
        pltpu.make_async_copy(
            tab_h.at[:, pl.ds(0, LANES)], tiles_v.at[0], fsem0).wait()
        total = process_column(cb0, 0, fsem0, total)
        @pl.when(cb1 < cb_hi)
        def _():
          @pl.when(cb1 + 1 < cb_hi)
          def _():
            nxt = pl.multiple_of((cb1 + 1) * LANES, LANES)
            pltpu.async_copy(
                tab_h.at[:, pl.ds(nxt, LANES)], tiles_v.at[0], fsem0)
          pltpu.make_async_copy(
              tab_h.at[:, pl.ds(0, LANES)], tiles_v.at[1], fsem1).wait()
        total = jnp.where(
            cb1 < cb_hi, process_column(cb1, 1, fsem1, total), total)
        return total

      npairs = (cb_hi - cb_lo + 1) // 2
      total = lax.fori_loop(0, npairs, per_pair, jnp.int32(0))

      for i in range(RING):
        @pl.when(i < jnp.minimum(total, RING))
        def _():
          pltpu.make_async_copy(
              out_o.at[pl.ds(0, F)],
              rows_v.at[pl.ds(i * F, F)], sem).wait()

      for h in bias_hs:
        h.wait()
      pltpu.async_copy(
          bias_v, bias_o.at[pl.ds(pl.multiple_of(base, BPW), BPW)], bsem).wait()

    do_table(us_h, tu_h, ur_o, bu_h, bug_o)
    do_table(mv_h, tm_h, mr_o, bm_h, bmg_o)

  return k(users, movies, tab_ut, tab_mt, bu_t, bm_t)


_HL2PI = 0.9189385332046727


def _stirling(z):
  r = 1.0 / z
  w = r * r
  series = r * (8.333333333333333e-2
                + w * (-2.777777777777778e-3 + w * 7.936507936507937e-4))
  return (z - 0.5) * jnp.log(z) - z + _HL2PI + series


def _lgamma_1(x):
  return _stirling(x + 3.0) - jnp.log(x * (x + 1.0) * (x + 2.0))


def _lgamma_2(s):
  return _stirling(s + 2.0) - jnp.log(s * (s + 1.0))


def _dg_series(z):
  r = 1.0 / z
  w = r * r
  return (jnp.log(z) - 0.5 * r
          - w * (8.333333333333333e-2
                 + w * (-8.333333333333333e-3 + w * 3.968253968253968e-3)))


def _digamma_1(x):
  num = 3.0 * x * x + 6.0 * x + 2.0
  den = x * (x + 1.0) * (x + 2.0)
  return _dg_series(x + 3.0) - num / den


def _digamma_2(s):
  return _dg_series(s + 2.0) - (2.0 * s + 1.0) / (s * (s + 1.0))


def _math_body(ur_ref, mr_ref, bug_ref, bmg_ref, out_ref):
  def fix(v):
    v = jnp.where(jnp.isnan(v), 0.05, v)
    return jnp.clip(v + 1.0, 1.0, 100.0)

  u = ur_ref[...]
  m = mr_ref[...]
  ub = jnp.concatenate([u[:, D:], u[:, :D]], axis=1)
  mb = jnp.concatenate([m[:, D:], m[:, :D]], axis=1)
  a1 = fix(u)
  b1 = fix(ub)
  a2 = fix(m)
  b2 = fix(mb)
  s1 = a1 + b1
  s2 = a2 + b2

  lnB1 = _lgamma_1(a1) + _lgamma_1(b1) - _lgamma_2(s1)
  lnB2 = _lgamma_1(a2) + _lgamma_1(b2) - _lgamma_2(s2)
  kl = (lnB2 - lnB1
        + (a1 - a2) * _digamma_1(a1)
        + (b1 - b2) * _digamma_1(b1)
        + (a2 - a1 + b2 - b1) * _digamma_2(s1))

  t = jnp.arctan2(jnp.abs(kl), 1.0) * (2.0 / jnp.pi)
  lane = lax.broadcasted_iota(jnp.int32, (BLK, LANES), 1)
  t = jnp.where(lane % F < D, t, 0.0)
  ri = lax.broadcasted_iota(jnp.int32, (LANES, EPR), 0)
  ci = lax.broadcasted_iota(jnp.int32, (LANES, EPR), 1)
  sel = jnp.where((ri // F) == ci, 1.0, 0.0).astype(jnp.float32)
  dist = jnp.dot(t, sel, preferred_element_type=jnp.float32)

  out_ref[...] = bug_ref[...] + bmg_ref[...] - dist


def _tc_math(ur, mr, bug, bmg):
  wide = pl.BlockSpec((BLK, LANES), lambda i: (i, 0))
  slim = pl.BlockSpec((BLK, EPR), lambda i: (i, 0))
  return pl.pallas_call(
      _math_body,
      grid=(GRID,),
      in_specs=[wide, wide, slim, slim],
      out_specs=slim,
      out_shape=jax.ShapeDtypeStruct((RB, EPR), jnp.float32),
  )(ur, mr, bug, bmg)


def kernel(x, u_table, m_table, Bu, Bm):
  users = x[:, 0].astype(jnp.int32)
  movies = x[:, 1].astype(jnp.int32)

  ur, mr, bug, bmg = _sc_gather(
      users, movies, u_table.T, m_table.T, Bu, Bm)

  out = _tc_math(ur.reshape(RB, LANES), mr.reshape(RB, LANES),
                 bug.reshape(RB, EPR), bmg.reshape(RB, EPR))
  return out.reshape(B)


# speedup vs baseline: 1.6437x; 1.6437x over previous
import functools

import jax
import jax.numpy as jnp
from jax import lax
from jax.experimental import pallas as pl
from jax.experimental.pallas import tpu as pltpu
from jax.experimental.pallas import tpu_sc as plsc

B = 16384
D = 16
NC = 2
NS = 16
NW = NC * NS
BPW = B // NW
CHUNK = 128
NCHUNK = BPW // CHUNK

LANES = 128
ROWS = B * D // LANES
GRID = 8
BLK = ROWS // GRID
GROUPS = LANES // D


def _sc_gather(users, movies, tab_u, tab_m, bu_t, bm_t):
  mesh = plsc.VectorSubcoreMesh(core_axis_name="c", subcore_axis_name="s")
  f32 = jnp.float32

  @functools.partial(
      pl.kernel,
      mesh=mesh,
      out_type=[
          jax.ShapeDtypeStruct((B * D,), f32),
          jax.ShapeDtypeStruct((B * D,), f32),
          jax.ShapeDtypeStruct((B * D,), f32),
          jax.ShapeDtypeStruct((B * D,), f32),
          jax.ShapeDtypeStruct((B,), f32),
          jax.ShapeDtypeStruct((B,), f32),
      ],
      scratch_types=[
          pltpu.VMEM((BPW,), jnp.int32),
          pltpu.VMEM((BPW,), jnp.int32),
          pltpu.VMEM((BPW * D,), f32),
          pltpu.VMEM((BPW * D,), f32),
          pltpu.VMEM((BPW * D,), f32),
          pltpu.VMEM((BPW * D,), f32),
          pltpu.VMEM((BPW,), f32),
          pltpu.VMEM((BPW,), f32),
          pltpu.SemaphoreType.DMA,
          pltpu.SemaphoreType.DMA,
      ],
  )
  def k(us_h, mv_h, tu_h, tm_h, bu_h, bm_h,
        au_o, buo_o, am_o, bmo_o, bug_o, bmg_o,
        us_v, mv_v, au_v, buv_v, am_v, bmv_v,
        bus_v, bms_v, isem, sem):
    wid = lax.axis_index("s") * NC + lax.axis_index("c")
    base = wid * BPW
    sl = pl.ds(base, BPW)

    hs = [pltpu.async_copy(us_h.at[sl], us_v, isem),
          pltpu.async_copy(mv_h.at[sl], mv_v, isem)]
    for h in hs:
      h.wait()

    bias_hs = []
    for tab, idx_v, dst_v in ((bu_h, us_v, bus_v), (bm_h, mv_v, bms_v)):
      for c in range(NCHUNK):
        cs = pl.ds(c * CHUNK, CHUNK)
        bias_hs.append(
            pltpu.async_copy(tab.at[idx_v.at[cs]], dst_v.at[cs], isem))

    def body(g, _):
      gsl = pl.ds(pl.multiple_of(g * 16, 16), 16)
      uvec = us_v[gsl]
      mvec = mv_v[gsl]
      for j in range(16):
        u = uvec[j]
        m = mvec[j]
        o = pl.multiple_of(g * (16 * D) + j * D, D)
        pltpu.async_copy(tu_h.at[u, pl.ds(0, D)], au_v.at[pl.ds(o, D)], sem)
        pltpu.async_copy(tu_h.at[u, pl.ds(D, D)], buv_v.at[pl.ds(o, D)], sem)
        pltpu.async_copy(tm_h.at[m, pl.ds(0, D)], am_v.at[pl.ds(o, D)], sem)
        pltpu.async_copy(tm_h.at[m, pl.ds(D, D)], bmv_v.at[pl.ds(o, D)], sem)
      return ()
    lax.fori_loop(0, BPW // 16, body, ())

    for buf in (au_v, buv_v, am_v, bmv_v):
      pltpu.make_async_copy(au_o.at[pl.ds(0, BPW * D)], buf, sem).wait()
    for h in bias_hs:
      h.wait()

    fsl = pl.ds(base * D, BPW * D)
    outs = [(au_v, au_o.at[fsl]), (buv_v, buo_o.at[fsl]),
            (am_v, am_o.at[fsl]), (bmv_v, bmo_o.at[fsl]),
            (bus_v, bug_o.at[sl]), (bms_v, bmg_o.at[sl])]
    hs = [pltpu.async_copy(src, dst, isem) for src, dst in outs]
    for h in hs:
      h.wait()

  return k(users, movies, tab_u, tab_m, bu_t, bm_t)


_HL2PI = 0.9189385332046727


def _stirling(z):
  r = 1.0 / z
  w = r * r
  series = r * (8.333333333333333e-2
                + w * (-2.777777777777778e-3 + w * 7.936507936507937e-4))
  return (z - 0.5) * jnp.log(z) - z + _HL2PI + series


def _lgamma_1(x):
  return _stirling(x + 3.0) - jnp.log(x * (x + 1.0) * (x + 2.0))


def _lgamma_2(s):
  return _stirling(s + 2.0) - jnp.log(s * (s + 1.0))


def _dg_series(z):
  r = 1.0 / z
  w = r * r
  return (jnp.log(z) - 0.5 * r
          - w * (8.333333333333333e-2
                 + w * (-8.333333333333333e-3 + w * 3.968253968253968e-3)))


def _digamma_1(x):
  num = 3.0 * x * x + 6.0 * x + 2.0
  den = x * (x + 1.0) * (x + 2.0)
  return _dg_series(x + 3.0) - num / den


def _digamma_2(s):
  return _dg_series(s + 2.0) - (2.0 * s + 1.0) / (s * (s + 1.0))


def _math_body(au_ref, bu_ref, am_ref, bm_ref, bug_ref, bmg_ref, out_ref):
  def fix(v):
    v = jnp.where(jnp.isnan(v), 0.05, v)
    return jnp.clip(v + 1.0, 1.0, 100.0)

  a1 = fix(au_ref[...])
  b1 = fix(bu_ref[...])
  a2 = fix(am_ref[...])
  b2 = fix(bm_ref[...])
  s1 = a1 + b1
  s2 = a2 + b2

  lnB1 = _lgamma_1(a1) + _lgamma_1(b1) - _lgamma_2(s1)
  lnB2 = _lgamma_1(a2) + _lgamma_1(b2) - _lgamma_2(s2)
  kl = (lnB2 - lnB1
        + (a1 - a2) * _digamma_1(a1)
        + (b1 - b2) * _digamma_1(b1)
        + (a2 - a1 + b2 - b1) * _digamma_2(s1))

  t = jnp.arctan2(jnp.abs(kl), 1.0) * (2.0 / jnp.pi)

  ri = lax.broadcasted_iota(jnp.int32, (LANES, GROUPS), 0)
  ci = lax.broadcasted_iota(jnp.int32, (LANES, GROUPS), 1)
  sel = jnp.where((ri // D) == ci, 1.0, 0.0).astype(jnp.float32)
  dist = jnp.dot(t, sel, preferred_element_type=jnp.float32)

  out_ref[...] = bug_ref[...] + bmg_ref[...] - dist


def _tc_math(au2, bu2, am2, bm2, bug2, bmg2):
  wide = pl.BlockSpec((BLK, LANES), lambda i: (i, 0))
  slim = pl.BlockSpec((BLK, GROUPS), lambda i: (i, 0))
  return pl.pallas_call(
      _math_body,
      grid=(GRID,),
      in_specs=[wide, wide, wide, wide, slim, slim],
      out_specs=slim,
      out_shape=jax.ShapeDtypeStruct((ROWS, GROUPS), jnp.float32),
  )(au2, bu2, am2, bm2, bug2, bmg2)


def kernel(x, u_table, m_table, Bu, Bm):
  users = x[:, 0].astype(jnp.int32)
  movies = x[:, 1].astype(jnp.int32)

  au, bu_, am, bm_, bug, bmg = _sc_gather(
      users, movies, u_table, m_table, Bu, Bm)

  out2 = _tc_math(
      au.reshape(ROWS, LANES), bu_.reshape(ROWS, LANES),
      am.reshape(ROWS, LANES), bm_.reshape(ROWS, LANES),
      bug.reshape(ROWS, GROUPS), bmg.reshape(ROWS, GROUPS))
  return out2.reshape(B)
